# speedup vs baseline: 1.0569x; 1.0569x over previous
import functools

import jax
import jax.numpy as jnp
import numpy as np
from jax.experimental import pallas as pl
from jax.experimental.pallas import tpu as pltpu

VOX_XY = 0.07
VOX_Z = 0.1
ALPHA = 5.0
EPS = 1e-5
PAD_KEY = np.int32(2**30)

_INTERPRET = False


def _mm_stats_body(g_ref, w_ref, m_ref, y_ref, st_ref):
    i = pl.program_id(0)
    y = jnp.dot(g_ref[...], w_ref[...], preferred_element_type=jnp.float32)
    y_ref[...] = y
    ym = y * m_ref[...]
    s1 = jnp.sum(ym, axis=0, keepdims=True)
    s2 = jnp.sum(ym * y, axis=0, keepdims=True)
    st = jnp.concatenate([s1, s2], axis=0)

    @pl.when(i == 0)
    def _():
        st_ref[...] = st

    @pl.when(i > 0)
    def _():
        st_ref[...] = st_ref[...] + st


def _mm_stats(g, w, rvf_col, bp=1000):
    p, k = g.shape
    co = w.shape[1]
    nb = p // bp
    return pl.pallas_call(
        _mm_stats_body,
        grid=(nb,),
        in_specs=[
            pl.BlockSpec((bp, k), lambda i: (i, 0)),
            pl.BlockSpec((k, co), lambda i: (0, 0)),
            pl.BlockSpec((bp, 1), lambda i: (i, 0)),
        ],
        out_specs=[
            pl.BlockSpec((bp, co), lambda i: (i, 0)),
            pl.BlockSpec((2, co), lambda i: (0, 0)),
        ],
        out_shape=[
            jax.ShapeDtypeStruct((p, co), jnp.float32),
            jax.ShapeDtypeStruct((2, co), jnp.float32),
        ],
        interpret=_INTERPRET,
    )(g, w, rvf_col)


def _bn_apply_body(y_ref, st_ref, cnt_ref, g_ref, b_ref, r_ref, h_ref):
    cnt = jnp.maximum(cnt_ref[0, 0], 1.0)
    m = st_ref[0:1, :] / cnt
    v = st_ref[1:2, :] / cnt - m * m
    scale = g_ref[...] * jax.lax.rsqrt(v + EPS)
    h = jnp.maximum((y_ref[...] - m) * scale + b_ref[...], 0.0)
    h_ref[...] = h * r_ref[...]


def _bn_apply(y, st, cnt11, g, b, rvf_col, bp=1000):
    p, co = y.shape
    nb = p // bp
    return pl.pallas_call(
        _bn_apply_body,
        grid=(nb,),
        in_specs=[
            pl.BlockSpec((bp, co), lambda i: (i, 0)),
            pl.BlockSpec((2, co), lambda i: (0, 0)),
            pl.BlockSpec((1, 1), lambda i: (0, 0)),
            pl.BlockSpec((1, co), lambda i: (0, 0)),
            pl.BlockSpec((1, co), lambda i: (0, 0)),
            pl.BlockSpec((bp, 1), lambda i: (i, 0)),
        ],
        out_specs=pl.BlockSpec((bp, co), lambda i: (i, 0)),
        out_shape=jax.ShapeDtypeStruct((p, co), jnp.float32),
        interpret=_INTERPRET,
    )(y, st, cnt11, g, b, rvf_col)


def _conv_layer(g_mat, w_flat, rvf_col, cnt11, gamma, beta):
    y, st = _mm_stats(g_mat, w_flat, rvf_col)
    return _bn_apply(y, st, cnt11, gamma[None, :], beta[None, :], rvf_col)


def _build_structure(xyzi, valid_mask):
    bt, _, n, _ = xyzi.shape
    p = bt * n
    xyz = jnp.transpose(xyzi[:, :3, :, 0], (0, 2, 1)).reshape(-1, 3)
    feat = jnp.transpose(xyzi[:, 3:, :, 0], (0, 2, 1)).reshape(-1, 4)
    vmask = valid_mask[:, 0, :, 0].reshape(-1)
    valid = vmask > 0.5

    x, y, z = xyz[:, 0], xyz[:, 1], xyz[:, 2]
    r = jnp.maximum(jnp.sqrt(x * x + y * y), 1e-06)
    w = ALPHA * jnp.log1p(r / ALPHA) / r
    qx = jnp.floor((x * w) / VOX_XY).astype(jnp.int32)
    qy = jnp.floor((y * w) / VOX_XY).astype(jnp.int32)
    qz = jnp.floor(z / VOX_Z).astype(jnp.int32)
    batch_idx = jnp.repeat(jnp.arange(bt, dtype=jnp.int32), n)

    coords_all = jnp.stack([batch_idx, qx, qy, qz], axis=1)
    sentinel = jnp.array([bt, 0, 0, 0], dtype=jnp.int32)
    coords = jnp.where(valid[:, None], coords_all, sentinel)

    cmin = coords.min(axis=0)
    c = coords - cmin + 1
    dims = c.max(axis=0) + 2
    d1, d2, d3 = dims[1], dims[2], dims[3]
    keys = ((c[:, 0] * d1 + c[:, 1]) * d2 + c[:, 2]) * d3 + c[:, 3]
    sent_c = sentinel - cmin + 1
    sent_key = ((sent_c[0] * d1 + sent_c[1]) * d2 + sent_c[2]) * d3 + sent_c[3]

    order = jnp.argsort(keys)
    skeys = keys[order]
    nf = jnp.concatenate(
        [jnp.ones((1,), jnp.int32), (skeys[1:] != skeys[:-1]).astype(jnp.int32)])
    runid = jnp.cumsum(nf) - 1

    ukeys = jnp.full((p,), PAD_KEY, jnp.int32).at[runid].set(skeys)
    counts = jnp.zeros((p,), jnp.int32).at[runid].add(1)
    inv = jnp.zeros((p,), jnp.int32).at[order].set(runid)
    rvf = ((ukeys != sent_key) & (ukeys != PAD_KEY)).astype(jnp.float32)
    cnt = rvf.sum()

    f0 = jnp.zeros((p, 4), jnp.float32).at[inv].add(feat)
    f0 = f0 / jnp.maximum(counts, 1)[:, None].astype(jnp.float32)
    f0 = f0 * rvf[:, None]

    nidx_l, nmask_l = [], []
    for dx in (-1, 0, 1):
        for dy in (-1, 0, 1):
            for dz in (-1, 0, 1):
                delta = (dx * d2 + dy) * d3 + dz
                nk = ukeys + delta
                pos = jnp.clip(jnp.searchsorted(ukeys, nk), 0, p - 1)
                found = ukeys[pos] == nk
                nidx_l.append(jnp.where(found, pos, 0).astype(jnp.int32))
                nmask_l.append(found.astype(jnp.float32))
    nidx = jnp.stack(nidx_l, axis=1)
    nmask = jnp.stack(nmask_l, axis=1)

    return feat, valid, inv, f0, rvf, cnt, nidx, nmask


def _gather_neighbors(h, nidx, nmask):
    p, c = h.shape
    g = jnp.take(h, nidx.reshape(-1), axis=0).reshape(p, 27, c)
    g = g * nmask[:, :, None]
    return g.reshape(p, 27 * c)


def kernel(xyzi, valid_mask, W1, g1, b1, W2, g2, b2, W3, g3, b3):
    bt, _, n, _ = xyzi.shape
    feat, valid, inv, f0, rvf, cnt, nidx, nmask = _build_structure(xyzi, valid_mask)

    rvf_col = rvf[:, None]
    cnt11 = cnt.reshape(1, 1)

    g1m = _gather_neighbors(f0, nidx, nmask)
    h1 = _conv_layer(g1m, W1.reshape(27 * 4, 32), rvf_col, cnt11, g1, b1)
    g2m = _gather_neighbors(h1, nidx, nmask)
    h2 = _conv_layer(g2m, W2.reshape(27 * 32, 64), rvf_col, cnt11, g2, b2)
    h3 = _conv_layer(h2, W3, rvf_col, cnt11, g3, b3)

    point_feats = jnp.take(h3, inv, axis=0)
    out_flat = jnp.where(valid[:, None], point_feats, 0.0)
    return jnp.transpose(out_flat.reshape(bt, n, 64), (0, 2, 1))[:, :, :, None]
